# baseline (speedup 1.0000x reference)
import functools

import jax
import jax.numpy as jnp
from jax import lax
from jax.experimental import pallas as pl
from jax.experimental.pallas import tpu as pltpu
from jax.experimental.pallas import tpu_sc as plsc

NC = 2
NS = 16
NW = NC * NS
L = 16

E_TOTAL = 320000
D = 128
W = D // 2
E_PER_W = E_TOTAL // NW
G = 80
STEPS = E_PER_W // G
GROUPS = G // L


def _sc_body(hs_hbm, src_hbm, dst_hbm, out_hbm,
             idx_s, idx_d, rows_s, rows_d, out_v, tbl_sp,
             sem_s0, sem_s1, sem_d0, sem_d1):
    sid = lax.axis_index("s")
    wid = sid * NC + lax.axis_index("c")
    sem_s = (sem_s0, sem_s1)
    sem_d = (sem_d0, sem_d1)

    @pl.when(sid == 0)
    def _stage_table():
        pltpu.sync_copy(hs_hbm, tbl_sp)

    pltpu.sync_copy(src_hbm.at[wid], idx_s)
    pltpu.sync_copy(dst_hbm.at[wid], idx_d)
    plsc.subcore_barrier()

    def issue_pair(step, b):
        pltpu.async_copy(tbl_sp.at[idx_s.at[step]], rows_s.at[b], sem_s[b])
        pltpu.async_copy(tbl_sp.at[idx_d.at[step]], rows_d.at[b], sem_d[b])

    def wait_pair(step, b):
        pltpu.make_async_copy(
            tbl_sp.at[idx_s.at[step]], rows_s.at[b], sem_s[b]).wait()
        pltpu.make_async_copy(
            tbl_sp.at[idx_d.at[step]], rows_d.at[b], sem_d[b]).wait()

    def compute(step, b):
        lane = lax.broadcasted_iota(jnp.int32, (L,), 0)
        off = [(lane + j) & 15 for j in range(16)]
        for g in range(GROUPS):
            e16 = g * L + lane
            z = jnp.zeros((L,), jnp.float32)

            def block(fo, accs):
                a = list(accs)
                base = fo * 16
                for j in range(16):
                    col = off[j] + base
                    s = plsc.load_gather(rows_s.at[b], [e16, col])
                    d = plsc.load_gather(rows_d.at[b], [e16, col])
                    p = plsc.bitcast(
                        plsc.bitcast(s, jnp.bfloat16)
                        * plsc.bitcast(d, jnp.bfloat16), jnp.int32)
                    p_lo = plsc.bitcast(p << 16, jnp.float32)
                    p_hi = plsc.bitcast(p & jnp.int32(-65536), jnp.float32)
                    a[(2 * j) % 8] = a[(2 * j) % 8] + p_lo
                    a[(2 * j + 1) % 8] = a[(2 * j + 1) % 8] + p_hi
                return tuple(a)

            accs = plsc.parallel_loop(
                0, W // 16, unroll=2, carry=(z,) * 8)(block)
            v = (((accs[0] + accs[1]) + (accs[2] + accs[3]))
                 + ((accs[4] + accs[5]) + (accs[6] + accs[7])))
            out_v[step, pl.ds(g * L, L)] = 1.0 / (1.0 + jnp.exp(-v))

    issue_pair(0, 0)

    def body2(i, carry):
        s0 = 2 * i
        issue_pair(s0 + 1, 1)
        wait_pair(s0, 0)
        compute(s0, 0)
        issue_pair(s0 + 2, 0)
        wait_pair(s0 + 1, 1)
        compute(s0 + 1, 1)
        return carry

    lax.fori_loop(0, (STEPS - 1) // 2, body2, 0)
    wait_pair(STEPS - 1, 0)
    compute(STEPS - 1, 0)
    pltpu.sync_copy(out_v, out_hbm.at[wid])


@jax.jit
def _decode(hidden_states, src_idx, dst_idx):
    mesh = plsc.VectorSubcoreMesh(core_axis_name="c", subcore_axis_name="s")
    f = pl.kernel(
        _sc_body,
        mesh=mesh,
        out_type=jax.ShapeDtypeStruct((NW, STEPS, G), jnp.float32),
        scratch_types=[
            pltpu.VMEM((STEPS, G), jnp.int32),
            pltpu.VMEM((STEPS, G), jnp.int32),
            pltpu.VMEM((2, G, W), jnp.int32),
            pltpu.VMEM((2, G, W), jnp.int32),
            pltpu.VMEM((STEPS, G), jnp.float32),
            pltpu.VMEM_SHARED((10000, W), jnp.int32),
            pltpu.SemaphoreType.DMA,
            pltpu.SemaphoreType.DMA,
            pltpu.SemaphoreType.DMA,
            pltpu.SemaphoreType.DMA,
        ],
        compiler_params=pltpu.CompilerParams(
            needs_layout_passes=False, use_tc_tiling_on_sc=False),
    )
    return f(hidden_states, src_idx, dst_idx)


def kernel(hidden_states, edge_index):
    ei = edge_index.astype(jnp.int32)
    src = ei[0].reshape(NW, STEPS, G)
    dst = ei[1].reshape(NW, STEPS, G)
    hs_bf16 = hidden_states.astype(jnp.bfloat16)
    hs_packed = jax.lax.bitcast_convert_type(
        hs_bf16.reshape(hidden_states.shape[0], W, 2), jnp.int32)
    out = _decode(hs_packed, src, dst)
    return out.reshape(E_TOTAL)
